# baseline (speedup 1.0000x reference)
import dataclasses
import functools

import jax
import jax.numpy as jnp
from jax import lax
from jax.experimental import pallas as pl
from jax.experimental.pallas import tpu as pltpu
from jax.experimental.pallas import tpu_sc as plsc

N = 10000
E = 320000
D = 128

NC = 2
NS = 16
NW = NC * NS
EPW = E // NW
CH = 112
NCH = 92
EPWP = NCH * CH
PAD = EPWP - EPW
RPT = 632
NP = RPT * NS
HR = 80
RB = 2000
B = 2
QB = 2 * B
GRPC = QB
NGRP = NCH // GRPC - 1


def _sc_agg_body(with_hist, x_hbm, ip_hbm, z_hbm, *refs):
    if with_hist:
        (sum_hbm, cnt_hbm, idx_v, rows_v, hist_v, acc_sh) = refs[:6]
        sems = refs[6:]
    else:
        (sum_hbm, idx_v, rows_v, acc_sh) = refs[:4]
        sems = refs[4:]
    gs = sems[:B]
    isems = sems[B:2 * B]
    ss = sems[2 * B:]
    c = lax.axis_index("c")
    s = lax.axis_index("s")
    wid = c * NS + s
    r0 = pl.multiple_of(s * RPT, 8)

    pltpu.sync_copy(z_hbm.at[pl.ds(r0, RPT)], acc_sh.at[pl.ds(r0, RPT)])
    if with_hist:
        pltpu.sync_copy(z_hbm.at[pl.ds(0, HR)], hist_v)
    plsc.subcore_barrier()

    for b in range(B):
        pltpu.sync_copy(ip_hbm.at[wid, b], idx_v.at[b])
        pltpu.async_copy(x_hbm.at[idx_v.at[b, 0]], rows_v.at[b], gs[b])

    if with_hist:
        ones16 = jnp.ones((16,), jnp.float32)

    def step(k, u, issue_next):
        b = u % B
        q = u % QB
        qn = (u + B) % QB
        pltpu.make_async_copy(
            x_hbm.at[idx_v.at[q, 0]], rows_v.at[b], gs[b]).wait()
        if issue_next:
            pltpu.async_copy(ip_hbm.at[wid, k + B], idx_v.at[qn], isems[b])
        pltpu.async_copy(rows_v.at[b], acc_sh.at[idx_v.at[q, 1]], ss[b],
                         add=True)
        if with_hist:
            for t in range(CH // 16):
                dv = idx_v[q, 1, pl.ds(t * 16, 16)]
                plsc.addupdate_scatter(
                    hist_v,
                    [lax.shift_right_logical(dv, 7),
                     lax.bitwise_and(dv, 127)], ones16)
        if issue_next:
            pltpu.make_async_copy(
                ip_hbm.at[wid, k + B], idx_v.at[qn], isems[b]).wait()
            pltpu.make_async_copy(
                rows_v.at[b], acc_sh.at[idx_v.at[q, 1]], ss[b]).wait()
            pltpu.async_copy(x_hbm.at[idx_v.at[qn, 0]], rows_v.at[b], gs[b])
        else:
            pltpu.make_async_copy(
                rows_v.at[b], acc_sh.at[idx_v.at[q, 1]], ss[b]).wait()

    @pl.loop(0, NGRP)
    def _(g):
        for u in range(GRPC):
            step(g * GRPC + u, u, True)

    base = NGRP * GRPC
    for u in range(B):
        step(base + u, u, True)
    for u in range(B, QB):
        step(base + u, u, False)

    plsc.subcore_barrier()
    pltpu.sync_copy(acc_sh.at[pl.ds(r0, RPT)], sum_hbm.at[c, pl.ds(r0, RPT)])
    if with_hist:
        pltpu.sync_copy(hist_v, cnt_hbm.at[wid])


_SC_MESH = plsc.VectorSubcoreMesh(core_axis_name="c", subcore_axis_name="s")

_SC_PARAMS = pltpu.CompilerParams()
if "needs_layout_passes" in pltpu.CompilerParams.__dataclass_fields__:
    _SC_PARAMS = dataclasses.replace(_SC_PARAMS, needs_layout_passes=False)

_sc_agg_hist = pl.kernel(
    functools.partial(_sc_agg_body, True),
    out_type=(jax.ShapeDtypeStruct((NC, NP, D), jnp.float32),
              jax.ShapeDtypeStruct((NW, HR, D), jnp.float32)),
    mesh=_SC_MESH,
    scratch_types=[
        pltpu.VMEM((QB, 2, CH), jnp.int32),
        pltpu.VMEM((B, CH, D), jnp.float32),
        pltpu.VMEM((HR, D), jnp.float32),
        pltpu.VMEM_SHARED((NP, D), jnp.float32),
    ] + [pltpu.SemaphoreType.DMA] * (3 * B),
    compiler_params=_SC_PARAMS,
)

_sc_agg = pl.kernel(
    functools.partial(_sc_agg_body, False),
    out_type=jax.ShapeDtypeStruct((NC, NP, D), jnp.float32),
    mesh=_SC_MESH,
    scratch_types=[
        pltpu.VMEM((QB, 2, CH), jnp.int32),
        pltpu.VMEM((B, CH, D), jnp.float32),
        pltpu.VMEM_SHARED((NP, D), jnp.float32),
    ] + [pltpu.SemaphoreType.DMA] * (3 * B),
)


def _tc_body(final, sa, sb, ch, x, wl, wr, b, o):
    cnt = jnp.maximum(jnp.sum(ch[:, :], axis=1), 1.0)[:, None]
    aggr = (sa[0] + sb[0]) / cnt
    h = (jnp.dot(aggr, wl[:, :], preferred_element_type=jnp.float32)
         + jnp.dot(x[:, :], wr[:, :], preferred_element_type=jnp.float32)
         + b[:, :])
    if final:
        m = jnp.max(h, axis=1, keepdims=True)
        lse = jnp.log(jnp.sum(jnp.exp(h - m), axis=1, keepdims=True)) + m
        o[:, :] = h - lse
    else:
        o[:, :] = jnp.maximum(h, 0.0)


def _make_tc(final):
    parta = pl.BlockSpec((1, RB, D), lambda i: (0, i, 0))
    partb = pl.BlockSpec((1, RB, D), lambda i: (1, i, 0))
    cnts = pl.BlockSpec((RB, NW), lambda i: (i, 0))
    row = pl.BlockSpec((RB, D), lambda i: (i, 0))
    full = pl.BlockSpec((D, D), lambda i: (0, 0))
    bias = pl.BlockSpec((1, D), lambda i: (0, 0))
    return pl.pallas_call(
        functools.partial(_tc_body, final),
        grid=(N // RB,),
        in_specs=[parta, partb, cnts, row, full, full, bias],
        out_specs=row,
        out_shape=jax.ShapeDtypeStruct((N, D), jnp.float32),
    )


_tc_relu = _make_tc(False)
_tc_logsm = _make_tc(True)


def _pad_edges(src, dst):
    wid = jnp.arange(NW, dtype=jnp.int32)[:, None]
    pad_i = jnp.arange(PAD, dtype=jnp.int32)[None, :]
    pad_src = jnp.broadcast_to((pad_i * 89) % N, (NW, PAD))
    pad_dst = N + (wid * 37 + pad_i) % (NP - N)
    srcp = jnp.concatenate([src.reshape(NW, EPW), pad_src], axis=1)
    dstp = jnp.concatenate([dst.reshape(NW, EPW), pad_dst], axis=1)
    return jnp.stack([srcp.reshape(NW, NCH, CH),
                      dstp.reshape(NW, NCH, CH)], axis=2)


def kernel(x, edge_index, W1_l, b1, W1_r, W2_l, b2, W2_r):
    src = edge_index[0].astype(jnp.int32)
    dst = edge_index[1].astype(jnp.int32)
    ip = _pad_edges(src, dst)
    zeros = jnp.zeros((NP, D), jnp.float32)

    sum1, cnth = _sc_agg_hist(x, ip, zeros)
    cnt = cnth.reshape(NW, HR * D)[:, :NP].T
    h = _tc_relu(sum1, sum1, cnt, x, W1_l.T, W1_r.T, b1[None, :])
    sum2 = _sc_agg(h, ip, zeros)
    return _tc_logsm(sum2, sum2, cnt, h, W2_l.T, W2_r.T, b2[None, :])


# speedup vs baseline: 1.0608x; 1.0608x over previous
import dataclasses
import functools

import jax
import jax.numpy as jnp
from jax import lax
from jax.experimental import pallas as pl
from jax.experimental.pallas import tpu as pltpu
from jax.experimental.pallas import tpu_sc as plsc

N = 10000
E = 320000
D = 128

NC = 2
NS = 16
NW = NC * NS
EPW = E // NW
CH = 128
NCH = 80
EPWP = NCH * CH
PAD = EPWP - EPW
RPT = 632
NP = RPT * NS
RB = 2000
B = 2
QB = 2 * B
GRPC = QB
NGRP = NCH // GRPC - 1


def _sc_agg_body(x_hbm, ip_hbm, z_hbm, sum_hbm, idx_v, rows_v, acc_sh,
                 *sems):
    gs = sems[:B]
    isems = sems[B:2 * B]
    ss = sems[2 * B:]
    c = lax.axis_index("c")
    s = lax.axis_index("s")
    wid = c * NS + s
    r0 = pl.multiple_of(s * RPT, 8)

    pltpu.sync_copy(z_hbm.at[pl.ds(r0, RPT)], acc_sh.at[pl.ds(r0, RPT)])
    plsc.subcore_barrier()

    for b in range(B):
        pltpu.sync_copy(ip_hbm.at[wid, b], idx_v.at[b])
        pltpu.async_copy(x_hbm.at[idx_v.at[b, 0]], rows_v.at[b], gs[b])

    def step(k, u, issue_next):
        b = u % B
        q = u % QB
        qn = (u + B) % QB
        pltpu.make_async_copy(
            x_hbm.at[idx_v.at[q, 0]], rows_v.at[b], gs[b]).wait()
        if issue_next:
            pltpu.async_copy(ip_hbm.at[wid, k + B], idx_v.at[qn], isems[b])
        pltpu.async_copy(rows_v.at[b], acc_sh.at[idx_v.at[q, 1]], ss[b],
                         add=True)
        if issue_next:
            pltpu.make_async_copy(
                ip_hbm.at[wid, k + B], idx_v.at[qn], isems[b]).wait()
            pltpu.make_async_copy(
                rows_v.at[b], acc_sh.at[idx_v.at[q, 1]], ss[b]).wait()
            pltpu.async_copy(x_hbm.at[idx_v.at[qn, 0]], rows_v.at[b], gs[b])
        else:
            pltpu.make_async_copy(
                rows_v.at[b], acc_sh.at[idx_v.at[q, 1]], ss[b]).wait()

    @pl.loop(0, NGRP)
    def _(g):
        for u in range(GRPC):
            step(g * GRPC + u, u, True)

    base = NGRP * GRPC
    for u in range(B):
        step(base + u, u, True)
    for u in range(B, QB):
        step(base + u, u, False)

    plsc.subcore_barrier()
    pltpu.sync_copy(acc_sh.at[pl.ds(r0, RPT)], sum_hbm.at[c, pl.ds(r0, RPT)])


def _sc_cnt_body(dstc_hbm, z1_hbm, cnt_hbm, dsti_v, hist_v):
    c = lax.axis_index("c")
    s = lax.axis_index("s")
    wid = c * NS + s

    pltpu.sync_copy(dstc_hbm.at[wid], dsti_v)
    pltpu.sync_copy(z1_hbm, hist_v)
    ones16 = jnp.ones((16,), jnp.float32)

    @pl.loop(0, EPWP, step=16)
    def _(i):
        dv = dsti_v[pl.ds(i, 16)]
        plsc.addupdate_scatter(hist_v, [dv], ones16)

    pltpu.sync_copy(hist_v, cnt_hbm.at[wid])


_SC_MESH = plsc.VectorSubcoreMesh(core_axis_name="c", subcore_axis_name="s")

_sc_agg = pl.kernel(
    _sc_agg_body,
    out_type=jax.ShapeDtypeStruct((NC, NP, D), jnp.float32),
    mesh=_SC_MESH,
    scratch_types=[
        pltpu.VMEM((QB, 2, CH), jnp.int32),
        pltpu.VMEM((B, CH, D), jnp.float32),
        pltpu.VMEM_SHARED((NP, D), jnp.float32),
    ] + [pltpu.SemaphoreType.DMA] * (3 * B),
)

_SC_CNT_PARAMS = pltpu.CompilerParams()
if "needs_layout_passes" in pltpu.CompilerParams.__dataclass_fields__:
    _SC_CNT_PARAMS = dataclasses.replace(
        _SC_CNT_PARAMS, needs_layout_passes=False)

_sc_cnt = pl.kernel(
    _sc_cnt_body,
    out_type=jax.ShapeDtypeStruct((NW, NP), jnp.float32),
    mesh=_SC_MESH,
    scratch_types=[
        pltpu.VMEM((EPWP,), jnp.int32),
        pltpu.VMEM((NP,), jnp.float32),
    ],
    compiler_params=_SC_CNT_PARAMS,
)


def _tc_body(final, sa, sb, ch, x, wl, wr, b, o):
    cnt = jnp.maximum(jnp.sum(ch[:, :], axis=1), 1.0)[:, None]
    aggr = (sa[0] + sb[0]) / cnt
    h = (jnp.dot(aggr, wl[:, :], preferred_element_type=jnp.float32)
         + jnp.dot(x[:, :], wr[:, :], preferred_element_type=jnp.float32)
         + b[:, :])
    if final:
        m = jnp.max(h, axis=1, keepdims=True)
        lse = jnp.log(jnp.sum(jnp.exp(h - m), axis=1, keepdims=True)) + m
        o[:, :] = h - lse
    else:
        o[:, :] = jnp.maximum(h, 0.0)


def _make_tc(final):
    parta = pl.BlockSpec((1, RB, D), lambda i: (0, i, 0))
    partb = pl.BlockSpec((1, RB, D), lambda i: (1, i, 0))
    cnts = pl.BlockSpec((RB, NW), lambda i: (i, 0))
    row = pl.BlockSpec((RB, D), lambda i: (i, 0))
    full = pl.BlockSpec((D, D), lambda i: (0, 0))
    bias = pl.BlockSpec((1, D), lambda i: (0, 0))
    return pl.pallas_call(
        functools.partial(_tc_body, final),
        grid=(N // RB,),
        in_specs=[parta, partb, cnts, row, full, full, bias],
        out_specs=row,
        out_shape=jax.ShapeDtypeStruct((N, D), jnp.float32),
    )


_tc_relu = _make_tc(False)
_tc_logsm = _make_tc(True)


def _pad_edges(src, dst):
    wid = jnp.arange(NW, dtype=jnp.int32)[:, None]
    pad_i = jnp.arange(PAD, dtype=jnp.int32)[None, :]
    pad_src = jnp.broadcast_to((pad_i * 89) % N, (NW, PAD))
    pad_dst = N + (wid * 37 + pad_i) % (NP - N)
    srcp = jnp.concatenate([src.reshape(NW, EPW), pad_src], axis=1)
    dstp = jnp.concatenate([dst.reshape(NW, EPW), pad_dst], axis=1)
    ip = jnp.stack([srcp.reshape(NW, NCH, CH),
                    dstp.reshape(NW, NCH, CH)], axis=2)
    return ip, dstp


def kernel(x, edge_index, W1_l, b1, W1_r, W2_l, b2, W2_r):
    src = edge_index[0].astype(jnp.int32)
    dst = edge_index[1].astype(jnp.int32)
    ip, dstc = _pad_edges(src, dst)
    zeros = jnp.zeros((NP, D), jnp.float32)

    cnt = _sc_cnt(dstc, jnp.zeros((NP,), jnp.float32)).T
    sum1 = _sc_agg(x, ip, zeros)
    h = _tc_relu(sum1, sum1, cnt, x, W1_l.T, W1_r.T, b1[None, :])
    sum2 = _sc_agg(h, ip, zeros)
    return _tc_logsm(sum2, sum2, cnt, h, W2_l.T, W2_r.T, b2[None, :])
